# baseline (speedup 1.0000x reference)
import functools

import jax
import jax.numpy as jnp
from jax import lax
from jax.experimental import pallas as pl
from jax.experimental.pallas import tpu as pltpu
from jax.experimental.pallas import tpu_sc as plsc

TOTAL_TOKENS = 1024
RETAIN = 256
BATCH = 64
C = 768

ROWS = BATCH * TOTAL_TOKENS
ROWS_Y = BATCH * RETAIN
NW = 32
RPW = ROWS // NW
Y_WORKERS = ROWS_Y // RPW
CHUNK = 32
NCH = RPW // CHUNK
NBUF = 4


def _build_sampler_kernel():
    info = plsc.get_sparse_core_info()
    nc = info.num_cores
    mesh = plsc.VectorSubcoreMesh(core_axis_name="c", subcore_axis_name="s")

    @functools.partial(
        pl.kernel,
        mesh=mesh,
        out_type=(
            jax.ShapeDtypeStruct((ROWS_Y, C), jnp.float32),
            jax.ShapeDtypeStruct((ROWS - ROWS_Y, C), jnp.float32),
        ),
        scratch_types=(
            [pltpu.VMEM((NCH, CHUNK), jnp.int32)]
            + [pltpu.VMEM((CHUNK, C), jnp.float32) for _ in range(NBUF)]
            + [pltpu.SemaphoreType.DMA for _ in range(2 * NBUF)]
        ),
    )
    def sampler(x_hbm, idx_hbm, y_hbm, z_hbm, idx_v, *bufs_and_sems):
        bufs = bufs_and_sems[:NBUF]
        gsem = bufs_and_sems[NBUF : 2 * NBUF]
        ssem = bufs_and_sems[2 * NBUF :]
        w = lax.axis_index("s") * nc + lax.axis_index("c")
        pltpu.sync_copy(idx_hbm.at[w], idx_v)

        def run(out_ref, obase):
            def gather(c, b):
                return pltpu.make_async_copy(
                    x_hbm.at[idx_v.at[c]], bufs[b], gsem[b]
                )

            def store(c, b):
                return pltpu.make_async_copy(
                    bufs[b],
                    out_ref.at[pl.ds(obase + c * CHUNK, CHUNK)],
                    ssem[b],
                )

            for b in range(NBUF - 1):
                gather(b, b).start()

            def body(i, carry):
                for b in range(NBUF):
                    cc = NBUF * i + b
                    gather(cc, b).wait()
                    store(cc, b).start()
                    nb = (b + NBUF - 1) % NBUF

                    @pl.when(cc + NBUF - 1 < NCH)
                    def _():
                        @pl.when(cc >= 1)
                        def _():
                            store(cc - 1, nb).wait()

                        gather(cc + NBUF - 1, nb).start()

                return carry

            lax.fori_loop(0, NCH // NBUF, body, 0)
            for b in range(NBUF):
                store(NCH - NBUF + b, (NCH - NBUF + b) % NBUF).wait()

        @pl.when(w < Y_WORKERS)
        def _():
            run(y_hbm, w * RPW)

        @pl.when(w >= Y_WORKERS)
        def _():
            run(z_hbm, (w - Y_WORKERS) * RPW)

    return sampler


_sampler = _build_sampler_kernel()


def kernel(x):
    perm = jax.random.permutation(jax.random.key(1), TOTAL_TOKENS)
    row_base = (jnp.arange(BATCH, dtype=jnp.int32) * TOTAL_TOKENS)[:, None]
    idx_y = (row_base + perm[None, :RETAIN]).reshape(-1)
    idx_z = (row_base + perm[None, RETAIN:]).reshape(-1)
    idx = (
        jnp.concatenate([idx_y, idx_z])
        .astype(jnp.int32)
        .reshape(NW, NCH, CHUNK)
    )
    y_flat, z_flat = _sampler(x.reshape(ROWS, C), idx)
    return (
        y_flat.reshape(BATCH, RETAIN, C),
        z_flat.reshape(BATCH, TOTAL_TOKENS - RETAIN, C),
    )
